# baseline (speedup 1.0000x reference)
import functools

import jax
import jax.numpy as jnp
from jax import lax
from jax.experimental import pallas as pl
from jax.experimental.pallas import tpu as pltpu

IN_DIM = 10
HID_DIM = 10
OUT_DIM = 5
LANES = 128

_W1_OFF = 0
_B1_OFF = 16
_W2_OFF = 24
_B2_OFF = 152

_CONTRACT0 = (((0,), (0,)), ((), ()))


_CHUNK = 16384


def _mlp_t_kernel(x_ref, p_ref, o_ref):
    w1 = p_ref[_W1_OFF:_W1_OFF + IN_DIM, :HID_DIM]
    b1 = p_ref[_B1_OFF:_B1_OFF + 1, :HID_DIM]
    w2 = p_ref[_W2_OFF:_W2_OFF + HID_DIM, :OUT_DIM]
    b2 = p_ref[_B2_OFF:_B2_OFF + 1, :OUT_DIM]

    tile = x_ref.shape[1]
    chunk = min(_CHUNK, tile)
    f32 = jnp.float32
    ones = jnp.ones((1, chunk), f32)
    for c in range(0, tile, chunk):
        x = x_ref[:, c:c + chunk]
        h = (lax.dot_general(w1, x, _CONTRACT0, preferred_element_type=f32)
             + lax.dot_general(b1, ones, _CONTRACT0, preferred_element_type=f32))
        h = jnp.maximum(h, 0.0)
        y = (lax.dot_general(w2, h, _CONTRACT0, preferred_element_type=f32)
             + lax.dot_general(b2, ones, _CONTRACT0, preferred_element_type=f32))
        o_ref[:, c:c + chunk] = y.astype(o_ref.dtype)


@functools.partial(jax.jit, static_argnames=("tile_b",))
def _forward(x, packed_params, *, tile_b=262144):
    B = x.shape[0]
    xt = x.T
    prows = packed_params.shape[0]

    tile = min(tile_b, ((B + LANES - 1) // LANES) * LANES)
    grid = (pl.cdiv(B, tile),)

    cost = pl.CostEstimate(
        flops=2 * B * (IN_DIM + 1) * (HID_DIM + OUT_DIM),
        transcendentals=0,
        bytes_accessed=B * (IN_DIM + OUT_DIM) * 4 + prows * LANES * 4,
    )

    yt = pl.pallas_call(
        _mlp_t_kernel,
        out_shape=jax.ShapeDtypeStruct((OUT_DIM, B), jnp.float32),
        grid=grid,
        in_specs=[
            pl.BlockSpec((IN_DIM, tile), lambda i: (0, i)),
            pl.BlockSpec((prows, LANES), lambda i: (0, 0)),
        ],
        out_specs=pl.BlockSpec((OUT_DIM, tile), lambda i: (0, i)),
        compiler_params=pltpu.CompilerParams(
            dimension_semantics=("parallel",),
        ),
        cost_estimate=cost,
    )(xt, packed_params)

    return yt.T


def kernel(x, packed_params):
    return _forward(x, packed_params)


# speedup vs baseline: 1.0455x; 1.0455x over previous
import functools

import jax
import jax.numpy as jnp
from jax import lax
from jax.experimental import pallas as pl
from jax.experimental.pallas import tpu as pltpu

IN_DIM = 10
HID_DIM = 10
OUT_DIM = 5
LANES = 128

_W1_OFF = 0
_B1_OFF = 16
_W2_OFF = 24
_B2_OFF = 152

_CONTRACT0 = (((0,), (0,)), ((), ()))


_CHUNK = 32768


def _mlp_t_kernel(x_ref, p_ref, o_ref):
    w1 = p_ref[_W1_OFF:_W1_OFF + IN_DIM, :HID_DIM]
    b1 = p_ref[_B1_OFF:_B1_OFF + 1, :HID_DIM]
    w2 = p_ref[_W2_OFF:_W2_OFF + HID_DIM, :OUT_DIM]
    b2 = p_ref[_B2_OFF:_B2_OFF + 1, :OUT_DIM]

    tile = x_ref.shape[1]
    chunk = min(_CHUNK, tile)
    f32 = jnp.float32
    ones = jnp.ones((1, chunk), f32)
    for c in range(0, tile, chunk):
        x = x_ref[:, c:c + chunk]
        h = (lax.dot_general(w1, x, _CONTRACT0, preferred_element_type=f32)
             + lax.dot_general(b1, ones, _CONTRACT0, preferred_element_type=f32))
        h = jnp.maximum(h, 0.0)
        y = (lax.dot_general(w2, h, _CONTRACT0, preferred_element_type=f32)
             + lax.dot_general(b2, ones, _CONTRACT0, preferred_element_type=f32))
        o_ref[:, c:c + chunk] = y.astype(o_ref.dtype)


@functools.partial(jax.jit, static_argnames=("tile_b",))
def _forward(x, packed_params, *, tile_b=262144):
    B = x.shape[0]
    xt = x.T
    prows = packed_params.shape[0]

    tile = min(tile_b, ((B + LANES - 1) // LANES) * LANES)
    grid = (pl.cdiv(B, tile),)

    cost = pl.CostEstimate(
        flops=2 * B * (IN_DIM + 1) * (HID_DIM + OUT_DIM),
        transcendentals=0,
        bytes_accessed=B * (IN_DIM + OUT_DIM) * 4 + prows * LANES * 4,
    )

    yt = pl.pallas_call(
        _mlp_t_kernel,
        out_shape=jax.ShapeDtypeStruct((OUT_DIM, B), jnp.float32),
        grid=grid,
        in_specs=[
            pl.BlockSpec((IN_DIM, tile), lambda i: (0, i)),
            pl.BlockSpec((prows, LANES), lambda i: (0, 0)),
        ],
        out_specs=pl.BlockSpec((OUT_DIM, tile), lambda i: (0, i)),
        compiler_params=pltpu.CompilerParams(
            dimension_semantics=("parallel",),
        ),
        cost_estimate=cost,
    )(xt, packed_params)

    return yt.T


def kernel(x, packed_params):
    return _forward(x, packed_params)
